# baseline (speedup 1.0000x reference)
import jax
import jax.numpy as jnp
from jax.experimental import pallas as pl
from jax.experimental.pallas import tpu as pltpu

N = 10000
E = 320000
B = 256
D = 128
ED = 16
H = 4
C = 128


def _gatv2(x, src, dst, edge_attr, Wl, bl, Wr, br, We, att, bias, heads, out_ch):
    n = x.shape[0]
    ones = jnp.ones((src.shape[0],), x.dtype)
    deg = jax.ops.segment_sum(ones, dst, num_segments=n)
    loop_attr = jax.ops.segment_sum(edge_attr, dst, num_segments=n) / jnp.maximum(deg, 1.0)[:, None]
    ar = jnp.arange(n, dtype=src.dtype)
    src2 = jnp.concatenate([src, ar])
    dst2 = jnp.concatenate([dst, ar])
    ea = jnp.concatenate([edge_attr, loop_attr], axis=0)
    xl = (x @ Wl + bl).reshape(n, heads, out_ch)
    xr = (x @ Wr + br).reshape(n, heads, out_ch)
    e = (ea @ We).reshape(-1, heads, out_ch)
    m = jax.nn.leaky_relu(xl[src2] + xr[dst2] + e, 0.2)
    logits = (m * att[None, :, :]).sum(-1)
    lmax = jax.ops.segment_max(logits, dst2, num_segments=n)
    lmax = jnp.where(jnp.isfinite(lmax), lmax, 0.0)
    ex = jnp.exp(logits - lmax[dst2])
    den = jax.ops.segment_sum(ex, dst2, num_segments=n)
    alpha = ex / (den[dst2] + 1e-16)
    out = jax.ops.segment_sum(alpha[:, :, None] * xl[src2], dst2, num_segments=n)
    return out.reshape(n, heads * out_ch) + bias


def _arm(x, edge_index, edge_attr, batch, p):
    src, dst = edge_index[0], edge_index[1]
    h = _gatv2(x, src, dst, edge_attr, p['Wl1'], p['bl1'], p['Wr1'], p['br1'], p['We1'], p['att1'], p['bias1'], H, C)
    h = jax.nn.elu(h)
    h = _gatv2(h, src, dst, edge_attr, p['Wl2'], p['bl2'], p['Wr2'], p['br2'], p['We2'], p['att2'], p['bias2'], 1, C)
    h = jax.nn.elu(h)
    cnt = jax.ops.segment_sum(jnp.ones((h.shape[0],), h.dtype), batch, num_segments=B)
    mean = jax.ops.segment_sum(h, batch, num_segments=B) / jnp.maximum(cnt, 1.0)[:, None]
    mx = jax.ops.segment_max(h, batch, num_segments=B)
    mx = jnp.where(jnp.isfinite(mx), mx, 0.0)
    return jnp.concatenate([mean, mx], axis=1)


def _mlp_kernel(va_ref, vb_ref, fc1_w_ref, fc1_b_ref, bn_s_ref, bn_o_ref,
                fc2_w_ref, fc2_b_ref, out_w_ref, out_b_ref, o_ref):
    va = va_ref[...]
    vb = vb_ref[...]
    combined = jnp.concatenate([va + vb, jnp.abs(va - vb), va * vb], axis=1)
    z = jnp.dot(combined, fc1_w_ref[...], preferred_element_type=jnp.float32) + fc1_b_ref[...]
    z = z * bn_s_ref[...] + bn_o_ref[...]
    z = jax.nn.relu(z)
    z = jax.nn.relu(jnp.dot(z, fc2_w_ref[...], preferred_element_type=jnp.float32) + fc2_b_ref[...])
    o_ref[...] = jnp.dot(z, out_w_ref[...], preferred_element_type=jnp.float32) + out_b_ref[...]


def kernel(x_a, edge_index_a, edge_attr_a, batch_a, x_b, edge_index_b, edge_attr_b, batch_b,
           Wl1, bl1, Wr1, br1, We1, att1, bias1, Wl2, bl2, Wr2, br2, We2, att2, bias2,
           fc1_w, fc1_b, bn_g, bn_b, bn_rm, bn_rv, fc2_w, fc2_b, out_w, out_b):
    p = dict(Wl1=Wl1, bl1=bl1, Wr1=Wr1, br1=br1, We1=We1, att1=att1, bias1=bias1,
             Wl2=Wl2, bl2=bl2, Wr2=Wr2, br2=br2, We2=We2, att2=att2, bias2=bias2)
    va = _arm(x_a, edge_index_a, edge_attr_a, batch_a, p)
    vb = _arm(x_b, edge_index_b, edge_attr_b, batch_b, p)
    bn_scale = bn_g / jnp.sqrt(bn_rv + 1e-5)
    bn_off = bn_b - bn_rm * bn_scale
    return pl.pallas_call(
        _mlp_kernel,
        out_shape=jax.ShapeDtypeStruct((B, 1), jnp.float32),
    )(va, vb, fc1_w, fc1_b, bn_scale, bn_off, fc2_w, fc2_b, out_w, out_b)


# speedup vs baseline: 8.5059x; 8.5059x over previous
import functools

import jax
import jax.numpy as jnp
from jax import lax
from jax.experimental import pallas as pl
from jax.experimental.pallas import tpu as pltpu
from jax.experimental.pallas import tpu_sc as plsc

N = 10000
E = 320000
B = 256
D = 128
ED = 16
H = 4
C = 128

NC = 2
NS = 16
NW = NC * NS

NP = 10240
EPT = E // NW
GA = 80
KA = EPT // GA

E2 = E + N
E2P = 331776
EPT2 = E2P // NW
G2 = 128
K2 = EPT2 // G2

_SC_PARAMS = pltpu.CompilerParams(use_tc_tiling_on_sc=False,
                                  needs_layout_passes=False)
_MESH = dict(core_axis_name="c", subcore_axis_name="s")


def _zero16():
    return jnp.zeros((16,), jnp.float32)


def _seg_attr_call(dst, ea):
    ones16 = jnp.zeros((GA, 16), jnp.float32).at[:, 0].set(1.0)
    RPS = NP // NS

    @functools.partial(
        pl.kernel,
        out_type=[jax.ShapeDtypeStruct((NC, NP, 16), jnp.float32),
                  jax.ShapeDtypeStruct((NC, NP, 16), jnp.float32)],
        mesh=plsc.VectorSubcoreMesh(**_MESH),
        compiler_params=_SC_PARAMS,
        scratch_types=[
            pltpu.VMEM((GA,), jnp.int32),
            pltpu.VMEM((GA, 16), jnp.float32),
            pltpu.VMEM((GA, 16), jnp.float32),
            pltpu.VMEM((RPS, 16), jnp.float32),
            pltpu.VMEM_SHARED((NP, 16), jnp.float32),
            pltpu.VMEM_SHARED((NP, 16), jnp.float32),
        ],
    )
    def k(dst_hbm, ea_hbm, ones_hbm, easum_hbm, deg_hbm,
          idx_v, val_v, ones_v, zv, acc_s, dacc_s):
        c = lax.axis_index("c")
        s = lax.axis_index("s")
        wid = c * NS + s
        zero = _zero16()

        def zfill(i, carry):
            zv[i, :] = zero
            return carry

        lax.fori_loop(0, RPS, zfill, 0)
        r0 = pl.multiple_of(s * RPS, RPS)
        pltpu.sync_copy(ones_hbm, ones_v)
        pltpu.sync_copy(zv, acc_s.at[pl.ds(r0, RPS)])
        pltpu.sync_copy(zv, dacc_s.at[pl.ds(r0, RPS)])
        plsc.subcore_barrier()

        def body(kk, carry):
            base = wid * EPT + kk * GA
            pltpu.sync_copy(dst_hbm.at[pl.ds(base, GA)], idx_v)
            pltpu.sync_copy(ea_hbm.at[pl.ds(base, GA)], val_v)
            pltpu.sync_copy(val_v, acc_s.at[idx_v], add=True)
            pltpu.sync_copy(ones_v, dacc_s.at[idx_v], add=True)
            return carry

        lax.fori_loop(0, KA, body, 0)
        plsc.subcore_barrier()
        pltpu.sync_copy(acc_s.at[pl.ds(r0, RPS)], zv)
        pltpu.sync_copy(zv, easum_hbm.at[c, pl.ds(r0, RPS)])
        pltpu.sync_copy(dacc_s.at[pl.ds(r0, RPS)], zv)
        pltpu.sync_copy(zv, deg_hbm.at[c, pl.ds(r0, RPS)])

    return k(dst, ea, ones16)


def _pass1_call(xl, xr, e, src2, dst2, att_flat, heads, hc_dim):
    G1 = 16384 // hc_dim
    K1 = EPT2 // G1
    NPH = NP * heads
    CHK = NPH // NS
    CPH16 = (hc_dim // heads) // 16

    @functools.partial(
        pl.kernel,
        out_type=[jax.ShapeDtypeStruct((heads, E2P), jnp.float32),
                  jax.ShapeDtypeStruct((NC, NPH), jnp.float32)],
        mesh=plsc.VectorSubcoreMesh(**_MESH),
        compiler_params=_SC_PARAMS,
        scratch_types=[
            pltpu.VMEM((G1,), jnp.int32),
            pltpu.VMEM((G1,), jnp.int32),
            pltpu.VMEM((G1, hc_dim), jnp.float32),
            pltpu.VMEM((G1, hc_dim), jnp.float32),
            pltpu.VMEM((G1, hc_dim), jnp.float32),
            pltpu.VMEM((hc_dim,), jnp.float32),
            pltpu.VMEM((heads * G1,), jnp.float32),
            pltpu.VMEM((heads * G1,), jnp.int32),
            pltpu.VMEM((CHK,), jnp.float32),
            pltpu.VMEM_SHARED((NPH,), jnp.float32),
            pltpu.SemaphoreType.DMA,
        ],
    )
    def k(xl_hbm, xr_hbm, e_hbm, src_hbm, dst_hbm, att_hbm, ex_hbm, den_hbm,
          sidx_v, didx_v, xlv, xrv, ev, attv, exv, fidx_v, zv, den_s, sem):
        c = lax.axis_index("c")
        s = lax.axis_index("s")
        wid = c * NS + s
        zero = _zero16()

        def zfill(i, carry):
            zv[pl.ds(i * 16, 16)] = zero
            return carry

        lax.fori_loop(0, CHK // 16, zfill, 0)
        r0 = pl.multiple_of(s * CHK, CHK)
        pltpu.sync_copy(att_hbm, attv)
        pltpu.sync_copy(zv, den_s.at[pl.ds(r0, CHK)])
        plsc.subcore_barrier()

        def body(kk, carry):
            base = wid * EPT2 + kk * G1
            pltpu.sync_copy(src_hbm.at[pl.ds(base, G1)], sidx_v)
            pltpu.sync_copy(dst_hbm.at[pl.ds(base, G1)], didx_v)
            d1 = pltpu.async_copy(xl_hbm.at[sidx_v], xlv, sem)
            d2 = pltpu.async_copy(xr_hbm.at[didx_v], xrv, sem)
            pltpu.sync_copy(e_hbm.at[pl.ds(base, G1)], ev)
            d1.wait()
            d2.wait()

            lane = lax.iota(jnp.int32, 16)

            def group(g, carry2):
                def edge16(ii, lvecs):
                    i = g * 16 + ii
                    sel = lane == ii
                    out = []
                    for h in range(heads):
                        acc = _zero16()
                        for jj in range(CPH16):
                            j = h * CPH16 + jj
                            v = (xlv[i, pl.ds(16 * j, 16)]
                                 + xrv[i, pl.ds(16 * j, 16)]
                                 + ev[i, pl.ds(16 * j, 16)])
                            m = jnp.maximum(v, 0.0) + 0.2 * jnp.minimum(v, 0.0)
                            acc = acc + m * attv[pl.ds(16 * j, 16)]
                        sc = jnp.full((16,), jnp.sum(acc), jnp.float32)
                        out.append(jnp.where(sel, sc, lvecs[h]))
                    return tuple(out)

                lvecs = lax.fori_loop(0, 16, edge16,
                                      tuple(_zero16() for _ in range(heads)))
                for h in range(heads):
                    exv[pl.ds(h * G1 + g * 16, 16)] = lvecs[h]
                return carry2

            lax.fori_loop(0, G1 // 16, group, 0)

            for w in range(heads * G1 // 16):
                x = exv[pl.ds(16 * w, 16)]
                exv[pl.ds(16 * w, 16)] = jnp.exp(jnp.minimum(x, 50.0))
                h = (16 * w) // G1
                sub = w - h * (G1 // 16)
                d16 = didx_v[pl.ds(16 * sub, 16)]
                fidx_v[pl.ds(16 * w, 16)] = d16 * heads + h
            for h in range(heads):
                pltpu.sync_copy(exv.at[pl.ds(h * G1, G1)],
                                ex_hbm.at[h, pl.ds(base, G1)])
            pltpu.sync_copy(exv, den_s.at[fidx_v], add=True)
            return carry

        lax.fori_loop(0, K1, body, 0)
        plsc.subcore_barrier()
        pltpu.sync_copy(den_s.at[pl.ds(r0, CHK)], zv)
        pltpu.sync_copy(zv, den_hbm.at[c, pl.ds(r0, CHK)])

    return k(xl, xr, e, src2, dst2, att_flat)


def _pass2_call(xlcb, src2, dst2, ex, deninv, heads, hc):
    NPH = NP * heads
    RP2 = NP // NS // G2

    @functools.partial(
        pl.kernel,
        out_type=jax.ShapeDtypeStruct((NC, NP, C), jnp.float32),
        mesh=plsc.VectorSubcoreMesh(**_MESH),
        compiler_params=_SC_PARAMS,
        scratch_types=[
            pltpu.VMEM((G2,), jnp.int32),
            pltpu.VMEM((G2,), jnp.int32),
            pltpu.VMEM((G2,), jnp.int32),
            pltpu.VMEM((G2,), jnp.int32),
            pltpu.VMEM((G2,), jnp.float32),
            pltpu.VMEM((G2,), jnp.float32),
            pltpu.VMEM((G2 + 16,), jnp.float32),
            pltpu.VMEM((G2, C), jnp.float32),
            pltpu.VMEM((G2, C), jnp.float32),
            pltpu.VMEM_SHARED((NP, C), jnp.float32),
            pltpu.SemaphoreType.DMA,
        ],
    )
    def k(xlcb_hbm, src_hbm, dst_hbm, ex_hbm, dinv_hbm, out_hbm,
          sidx_v, didx_v, gidx_v, didx2_v, exq, dq, aq, xcv, cv, out_s, sem):
        c = lax.axis_index("c")
        s = lax.axis_index("s")
        wid = c * NS + s
        zero = _zero16()

        def zfill(i, carry):
            for j in range(C // 16):
                cv[i, pl.ds(16 * j, 16)] = zero
            return carry

        lax.fori_loop(0, G2, zfill, 0)
        r0 = pl.multiple_of(s * (NP // NS), NP // NS)
        for t in range(RP2):
            pltpu.sync_copy(cv, out_s.at[pl.ds(r0 + G2 * t, G2)])
        plsc.subcore_barrier()

        def body(kk, carry):
            base = wid * EPT2 + kk * G2
            pltpu.sync_copy(src_hbm.at[pl.ds(base, G2)], sidx_v)
            pltpu.sync_copy(dst_hbm.at[pl.ds(base, G2)], didx_v)
            for w in range(G2 // 16):
                gidx_v[pl.ds(16 * w, 16)] = sidx_v[pl.ds(16 * w, 16)] + hc * N
                didx2_v[pl.ds(16 * w, 16)] = didx_v[pl.ds(16 * w, 16)] * heads + hc
            d1 = pltpu.async_copy(xlcb_hbm.at[gidx_v], xcv, sem)
            d2 = pltpu.async_copy(dinv_hbm.at[didx2_v], dq, sem)
            pltpu.sync_copy(ex_hbm.at[hc, pl.ds(base, G2)], exq)
            d1.wait()
            d2.wait()
            for w in range(G2 // 16):
                aq[pl.ds(16 * w, 16)] = exq[pl.ds(16 * w, 16)] * dq[pl.ds(16 * w, 16)]

            def edge(i, carry2):
                av = jnp.full((16,), aq[pl.ds(i, 16)][0], jnp.float32)
                for j in range(C // 16):
                    cv[i, pl.ds(16 * j, 16)] = xcv[i, pl.ds(16 * j, 16)] * av
                return carry2

            lax.fori_loop(0, G2, edge, 0)
            pltpu.sync_copy(cv, out_s.at[didx_v], add=True)
            return carry

        lax.fori_loop(0, K2, body, 0)
        plsc.subcore_barrier()
        for t in range(RP2):
            pltpu.sync_copy(out_s.at[pl.ds(r0 + G2 * t, G2)], cv)
            pltpu.sync_copy(cv, out_hbm.at[c, pl.ds(r0 + G2 * t, G2)])

    return k(xlcb, src2, dst2, ex, deninv)


def _dual_mm_kernel(x_ref, wl_ref, bl_ref, wr_ref, br_ref, o1_ref, o2_ref):
    xb = x_ref[...]
    o1_ref[...] = jnp.dot(xb, wl_ref[...], preferred_element_type=jnp.float32) + bl_ref[...]
    o2_ref[...] = jnp.dot(xb, wr_ref[...], preferred_element_type=jnp.float32) + br_ref[...]


def _dual_mm(x, wl, bl, wr, br, bm):
    m, kdim = x.shape
    n = wl.shape[1]
    return pl.pallas_call(
        _dual_mm_kernel,
        grid=(m // bm,),
        in_specs=[pl.BlockSpec((bm, kdim), lambda i: (i, 0)),
                  pl.BlockSpec((kdim, n), lambda i: (0, 0)),
                  pl.BlockSpec((n,), lambda i: (0,)),
                  pl.BlockSpec((kdim, n), lambda i: (0, 0)),
                  pl.BlockSpec((n,), lambda i: (0,))],
        out_specs=[pl.BlockSpec((bm, n), lambda i: (i, 0)),
                   pl.BlockSpec((bm, n), lambda i: (i, 0))],
        out_shape=[jax.ShapeDtypeStruct((m, n), jnp.float32),
                   jax.ShapeDtypeStruct((m, n), jnp.float32)],
    )(x, wl, bl, wr, br)


def _e_mm_kernel(ea_ref, we_ref, o_ref):
    o_ref[...] = jnp.dot(ea_ref[...], we_ref[...], preferred_element_type=jnp.float32)


def _e_mm(ea2, we):
    m = ea2.shape[0]
    n = we.shape[1]
    bm = 4096
    return pl.pallas_call(
        _e_mm_kernel,
        grid=(m // bm,),
        in_specs=[pl.BlockSpec((bm, ED), lambda i: (i, 0)),
                  pl.BlockSpec((ED, n), lambda i: (0, 0))],
        out_specs=pl.BlockSpec((bm, n), lambda i: (i, 0)),
        out_shape=jax.ShapeDtypeStruct((m, n), jnp.float32),
    )(ea2, we)


def _deninv_kernel(d_ref, o_ref):
    o_ref[...] = 1.0 / (d_ref[0] + d_ref[1] + 1e-16)


def _deninv(dden):
    nph = dden.shape[1]
    d3 = dden.reshape(NC, nph // 128, 128)
    out = pl.pallas_call(
        _deninv_kernel,
        out_shape=jax.ShapeDtypeStruct((nph // 128, 128), jnp.float32),
    )(d3)
    return out.reshape(nph)


def _combine_elu_kernel(p_ref, b_ref, o_ref):
    z = p_ref[0, 0] + p_ref[0, 1] + b_ref[0]
    o_ref[0] = jnp.where(z > 0, z, jnp.exp(z) - 1.0)


def _combine_elu(parts, bias_hc, heads):
    bn = 512
    return pl.pallas_call(
        _combine_elu_kernel,
        grid=(heads, NP // bn),
        in_specs=[pl.BlockSpec((1, NC, bn, C), lambda h, i: (h, 0, i, 0)),
                  pl.BlockSpec((1, 1, C), lambda h, i: (h, 0, 0))],
        out_specs=pl.BlockSpec((1, bn, C), lambda h, i: (h, i, 0)),
        out_shape=jax.ShapeDtypeStruct((heads, NP, C), jnp.float32),
    )(parts, bias_hc.reshape(heads, 1, C))


def _l2_mm_kernel(h_ref, wl_ref, bl_ref, wr_ref, br_ref, o1_ref, o2_ref):
    accl = jnp.zeros(o1_ref.shape, jnp.float32)
    accr = jnp.zeros(o2_ref.shape, jnp.float32)
    for h in range(H):
        hb = h_ref[h]
        accl = accl + jnp.dot(hb, wl_ref[h], preferred_element_type=jnp.float32)
        accr = accr + jnp.dot(hb, wr_ref[h], preferred_element_type=jnp.float32)
    o1_ref[...] = accl + bl_ref[...]
    o2_ref[...] = accr + br_ref[...]


def _l2_mm(h_chunks, wl2, bl2, wr2, br2):
    bn = 512
    wl3 = wl2.reshape(H, C, C)
    wr3 = wr2.reshape(H, C, C)
    return pl.pallas_call(
        _l2_mm_kernel,
        grid=(NP // bn,),
        in_specs=[pl.BlockSpec((H, bn, C), lambda i: (0, i, 0)),
                  pl.BlockSpec((H, C, C), lambda i: (0, 0, 0)),
                  pl.BlockSpec((C,), lambda i: (0,)),
                  pl.BlockSpec((H, C, C), lambda i: (0, 0, 0)),
                  pl.BlockSpec((C,), lambda i: (0,))],
        out_specs=[pl.BlockSpec((bn, C), lambda i: (i, 0)),
                   pl.BlockSpec((bn, C), lambda i: (i, 0))],
        out_shape=[jax.ShapeDtypeStruct((NP, C), jnp.float32),
                   jax.ShapeDtypeStruct((NP, C), jnp.float32)],
    )(h_chunks, wl3, bl2, wr3, br2)


def _mlp_kernel(va_ref, vb_ref, fc1_w_ref, fc1_b_ref, bn_s_ref, bn_o_ref,
                fc2_w_ref, fc2_b_ref, out_w_ref, out_b_ref, o_ref):
    va = va_ref[...]
    vb = vb_ref[...]
    combined = jnp.concatenate([va + vb, jnp.abs(va - vb), va * vb], axis=1)
    z = jnp.dot(combined, fc1_w_ref[...], preferred_element_type=jnp.float32) + fc1_b_ref[...]
    z = z * bn_s_ref[...] + bn_o_ref[...]
    z = jax.nn.relu(z)
    z = jax.nn.relu(jnp.dot(z, fc2_w_ref[...], preferred_element_type=jnp.float32) + fc2_b_ref[...])
    o_ref[...] = jnp.dot(z, out_w_ref[...], preferred_element_type=jnp.float32) + out_b_ref[...]


def _gat_layer(xpad, src2, dst2, e_full, att, bias, wl, bl, wr, br, heads):
    hc_dim = heads * C
    xl, xr = _dual_mm(xpad, wl, bl, wr, br, bm=512)
    att_flat = att.reshape(hc_dim)
    ex, dden = _pass1_call(xl, xr, e_full, src2, dst2, att_flat, heads, hc_dim)
    dinv = _deninv(dden)
    if heads == 1:
        xlcb = xl
    else:
        xlcb = xl[:N].reshape(N, heads, C).transpose(1, 0, 2).reshape(heads * N, C)
    parts = []
    for hc in range(heads):
        parts.append(_pass2_call(xlcb, src2, dst2, ex, dinv, heads, hc))
    parts = jnp.stack(parts)
    return _combine_elu(parts, bias.reshape(heads, C), heads)


def _arm(x, edge_index, edge_attr, batch, p):
    src, dst = edge_index[0], edge_index[1]
    easum, deg16 = _seg_attr_call(dst, edge_attr)
    deg = deg16[0, :N, 0] + deg16[1, :N, 0]
    loop_attr = (easum[0, :N] + easum[1, :N]) / jnp.maximum(deg, 1.0)[:, None]

    ar = jnp.arange(N, dtype=jnp.int32)
    npad = E2P - E2
    src2 = jnp.concatenate([src, ar, jnp.zeros((npad,), jnp.int32)])
    dst2 = jnp.concatenate([dst, ar, jnp.full((npad,), N, jnp.int32)])
    ea2 = jnp.concatenate([edge_attr, loop_attr, jnp.zeros((npad, ED), jnp.float32)], axis=0)
    xpad = jnp.pad(x, ((0, NP - N), (0, 0)))

    e1 = _e_mm(ea2, p['We1'])
    h1 = _gat_layer(xpad, src2, dst2, e1, p['att1'], p['bias1'],
                    p['Wl1'], p['bl1'], p['Wr1'], p['br1'], H)

    xl2, xr2 = _l2_mm(h1, p['Wl2'], p['bl2'], p['Wr2'], p['br2'])
    e2 = _e_mm(ea2, p['We2'])
    att2_flat = p['att2'].reshape(C)
    ex2, dden2 = _pass1_call(xl2, xr2, e2, src2, dst2, att2_flat, 1, C)
    dinv2 = _deninv(dden2)
    part2 = _pass2_call(xl2, src2, dst2, ex2, dinv2, 1, 0)
    h2 = _combine_elu(part2[None], p['bias2'].reshape(1, C), 1)[0]

    hN = h2[:N]
    cnt = jax.ops.segment_sum(jnp.ones((N,), hN.dtype), batch, num_segments=B)
    mean = jax.ops.segment_sum(hN, batch, num_segments=B) / jnp.maximum(cnt, 1.0)[:, None]
    mx = jax.ops.segment_max(hN, batch, num_segments=B)
    mx = jnp.where(jnp.isfinite(mx), mx, 0.0)
    return jnp.concatenate([mean, mx], axis=1)


def kernel(x_a, edge_index_a, edge_attr_a, batch_a, x_b, edge_index_b, edge_attr_b, batch_b,
           Wl1, bl1, Wr1, br1, We1, att1, bias1, Wl2, bl2, Wr2, br2, We2, att2, bias2,
           fc1_w, fc1_b, bn_g, bn_b, bn_rm, bn_rv, fc2_w, fc2_b, out_w, out_b):
    p = dict(Wl1=Wl1, bl1=bl1, Wr1=Wr1, br1=br1, We1=We1, att1=att1, bias1=bias1,
             Wl2=Wl2, bl2=bl2, Wr2=Wr2, br2=br2, We2=We2, att2=att2, bias2=bias2)
    va = _arm(x_a, edge_index_a, edge_attr_a, batch_a, p)
    vb = _arm(x_b, edge_index_b, edge_attr_b, batch_b, p)
    bn_scale = bn_g / jnp.sqrt(bn_rv + 1e-5)
    bn_off = bn_b - bn_rm * bn_scale
    return pl.pallas_call(
        _mlp_kernel,
        out_shape=jax.ShapeDtypeStruct((B, 1), jnp.float32),
    )(va, vb, fc1_w, fc1_b, bn_scale, bn_off, fc2_w, fc2_b, out_w, out_b)


# speedup vs baseline: 10.1424x; 1.1924x over previous
import functools

import jax
import jax.numpy as jnp
from jax import lax
from jax.experimental import pallas as pl
from jax.experimental.pallas import tpu as pltpu
from jax.experimental.pallas import tpu_sc as plsc

N = 10000
E = 320000
B = 256
D = 128
ED = 16
H = 4
C = 128

NC = 2
NS = 16
NW = NC * NS

NP = 10240
EPT = E // NW
GA = 80
KA = EPT // GA

E2 = E + N
E2P = 331776
EPT2 = E2P // NW
G2 = 128
K2 = EPT2 // G2

_SC_PARAMS = pltpu.CompilerParams(use_tc_tiling_on_sc=False,
                                  needs_layout_passes=False)
_MESH = dict(core_axis_name="c", subcore_axis_name="s")


def _zero16():
    return jnp.zeros((16,), jnp.float32)


def _seg_attr_call(dst, ea):
    ones16 = jnp.zeros((GA, 16), jnp.float32).at[:, 0].set(1.0)
    RPS = NP // NS

    @functools.partial(
        pl.kernel,
        out_type=[jax.ShapeDtypeStruct((NC, NP, 16), jnp.float32),
                  jax.ShapeDtypeStruct((NC, NP, 16), jnp.float32)],
        mesh=plsc.VectorSubcoreMesh(**_MESH),
        compiler_params=_SC_PARAMS,
        scratch_types=[
            pltpu.VMEM((GA,), jnp.int32),
            pltpu.VMEM((GA, 16), jnp.float32),
            pltpu.VMEM((GA, 16), jnp.float32),
            pltpu.VMEM((RPS, 16), jnp.float32),
            pltpu.VMEM_SHARED((NP, 16), jnp.float32),
            pltpu.VMEM_SHARED((NP, 16), jnp.float32),
        ],
    )
    def k(dst_hbm, ea_hbm, ones_hbm, easum_hbm, deg_hbm,
          idx_v, val_v, ones_v, zv, acc_s, dacc_s):
        c = lax.axis_index("c")
        s = lax.axis_index("s")
        wid = c * NS + s
        zero = _zero16()

        def zfill(i, carry):
            zv[i, :] = zero
            return carry

        lax.fori_loop(0, RPS, zfill, 0)
        r0 = pl.multiple_of(s * RPS, RPS)
        pltpu.sync_copy(ones_hbm, ones_v)
        pltpu.sync_copy(zv, acc_s.at[pl.ds(r0, RPS)])
        pltpu.sync_copy(zv, dacc_s.at[pl.ds(r0, RPS)])
        plsc.subcore_barrier()

        def body(kk, carry):
            base = wid * EPT + kk * GA
            pltpu.sync_copy(dst_hbm.at[pl.ds(base, GA)], idx_v)
            pltpu.sync_copy(ea_hbm.at[pl.ds(base, GA)], val_v)
            pltpu.sync_copy(val_v, acc_s.at[idx_v], add=True)
            pltpu.sync_copy(ones_v, dacc_s.at[idx_v], add=True)
            return carry

        lax.fori_loop(0, KA, body, 0)
        plsc.subcore_barrier()
        pltpu.sync_copy(acc_s.at[pl.ds(r0, RPS)], zv)
        pltpu.sync_copy(zv, easum_hbm.at[c, pl.ds(r0, RPS)])
        pltpu.sync_copy(dacc_s.at[pl.ds(r0, RPS)], zv)
        pltpu.sync_copy(zv, deg_hbm.at[c, pl.ds(r0, RPS)])

    return k(dst, ea, ones16)


def _pass1_call(xl, xr, e, src2, dst2, att_flat, heads, hc_dim):
    G1 = 32 if hc_dim >= 512 else 64
    K1 = EPT2 // G1
    NPH = NP * heads
    CHK = NPH // NS
    CPH16 = (hc_dim // heads) // 16

    @functools.partial(
        pl.kernel,
        out_type=[jax.ShapeDtypeStruct((heads, E2P), jnp.float32),
                  jax.ShapeDtypeStruct((NC, NPH), jnp.float32)],
        mesh=plsc.VectorSubcoreMesh(**_MESH),
        compiler_params=_SC_PARAMS,
        scratch_types=[
            pltpu.VMEM((2, G1), jnp.int32),
            pltpu.VMEM((2, G1), jnp.int32),
            pltpu.VMEM((2, G1, hc_dim), jnp.float32),
            pltpu.VMEM((2, G1, hc_dim), jnp.float32),
            pltpu.VMEM((2, G1, hc_dim), jnp.float32),
            pltpu.VMEM((hc_dim,), jnp.float32),
            pltpu.VMEM((heads * G1,), jnp.float32),
            pltpu.VMEM((heads * G1,), jnp.int32),
            pltpu.VMEM((CHK,), jnp.float32),
            pltpu.VMEM_SHARED((NPH,), jnp.float32),
            pltpu.SemaphoreType.DMA,
            pltpu.SemaphoreType.DMA,
        ],
    )
    def k(xl_hbm, xr_hbm, e_hbm, src_hbm, dst_hbm, att_hbm, ex_hbm, den_hbm,
          sidx_v, didx_v, xlv, xrv, ev, attv, exv, fidx_v, zv, den_s,
          sem0, sem1):
        c = lax.axis_index("c")
        s = lax.axis_index("s")
        wid = c * NS + s
        zero = _zero16()
        sems = (sem0, sem1)

        def zfill(i, carry):
            zv[pl.ds(i * 16, 16)] = zero
            return carry

        lax.fori_loop(0, CHK // 16, zfill, 0)
        r0 = pl.multiple_of(s * CHK, CHK)
        pltpu.sync_copy(att_hbm, attv)
        pltpu.sync_copy(zv, den_s.at[pl.ds(r0, CHK)])
        plsc.subcore_barrier()

        def issue(kk, b):
            base = wid * EPT2 + kk * G1
            pltpu.sync_copy(src_hbm.at[pl.ds(base, G1)], sidx_v.at[b])
            pltpu.sync_copy(dst_hbm.at[pl.ds(base, G1)], didx_v.at[b])
            pltpu.async_copy(xl_hbm.at[sidx_v.at[b]], xlv.at[b], sems[b])
            pltpu.async_copy(xr_hbm.at[didx_v.at[b]], xrv.at[b], sems[b])
            pltpu.async_copy(e_hbm.at[pl.ds(base, G1)], ev.at[b], sems[b])

        def wait(b):
            pltpu.make_async_copy(xl_hbm.at[sidx_v.at[b]], xlv.at[b], sems[b]).wait()
            pltpu.make_async_copy(xr_hbm.at[didx_v.at[b]], xrv.at[b], sems[b]).wait()
            pltpu.make_async_copy(e_hbm.at[pl.ds(0, G1)], ev.at[b], sems[b]).wait()

        lane = lax.iota(jnp.int32, 16)

        def compute(kk, b):
            base = wid * EPT2 + kk * G1

            def group(g, carry2):
                def edge16(ii, lvecs):
                    i = g * 16 + ii
                    sel = lane == ii
                    out = []
                    for h in range(heads):
                        acc = _zero16()
                        for jj in range(CPH16):
                            j = h * CPH16 + jj
                            v = (xlv[b, i, pl.ds(16 * j, 16)]
                                 + xrv[b, i, pl.ds(16 * j, 16)]
                                 + ev[b, i, pl.ds(16 * j, 16)])
                            m = jnp.maximum(v, 0.0) + 0.2 * jnp.minimum(v, 0.0)
                            acc = acc + m * attv[pl.ds(16 * j, 16)]
                        sc = jnp.full((16,), jnp.sum(acc), jnp.float32)
                        out.append(jnp.where(sel, sc, lvecs[h]))
                    return tuple(out)

                lvecs = lax.fori_loop(0, 16, edge16,
                                      tuple(_zero16() for _ in range(heads)))
                for h in range(heads):
                    exv[pl.ds(h * G1 + g * 16, 16)] = lvecs[h]
                return carry2

            lax.fori_loop(0, G1 // 16, group, 0)

            for w in range(heads * G1 // 16):
                x = exv[pl.ds(16 * w, 16)]
                exv[pl.ds(16 * w, 16)] = jnp.exp(jnp.minimum(x, 50.0))
                h = (16 * w) // G1
                sub = w - h * (G1 // 16)
                d16 = didx_v[b, pl.ds(16 * sub, 16)]
                fidx_v[pl.ds(16 * w, 16)] = d16 * heads + h
            for h in range(heads):
                pltpu.sync_copy(exv.at[pl.ds(h * G1, G1)],
                                ex_hbm.at[h, pl.ds(base, G1)])
            pltpu.sync_copy(exv, den_s.at[fidx_v], add=True)

        issue(0, 0)
        issue(1, 1)

        def outer(g, carry):
            for b in range(2):
                kk = 2 * g + b
                wait(b)
                compute(kk, b)

                @pl.when(kk + 2 < K1)
                def _():
                    issue(kk + 2, b)
            return carry

        lax.fori_loop(0, K1 // 2, outer, 0)
        plsc.subcore_barrier()
        pltpu.sync_copy(den_s.at[pl.ds(r0, CHK)], zv)
        pltpu.sync_copy(zv, den_hbm.at[c, pl.ds(r0, CHK)])

    return k(xl, xr, e, src2, dst2, att_flat)


def _pass2_call(xlcb, src2, dst2, ex, deninv, heads, hc):
    NPH = NP * heads
    RP2 = NP // NS // G2

    @functools.partial(
        pl.kernel,
        out_type=jax.ShapeDtypeStruct((NC, NP, C), jnp.float32),
        mesh=plsc.VectorSubcoreMesh(**_MESH),
        compiler_params=_SC_PARAMS,
        scratch_types=[
            pltpu.VMEM((G2,), jnp.int32),
            pltpu.VMEM((G2,), jnp.int32),
            pltpu.VMEM((G2,), jnp.int32),
            pltpu.VMEM((G2,), jnp.int32),
            pltpu.VMEM((G2,), jnp.float32),
            pltpu.VMEM((G2,), jnp.float32),
            pltpu.VMEM((G2 + 16,), jnp.float32),
            pltpu.VMEM((G2, C), jnp.float32),
            pltpu.VMEM((G2, C), jnp.float32),
            pltpu.VMEM_SHARED((NP, C), jnp.float32),
            pltpu.SemaphoreType.DMA,
        ],
    )
    def k(xlcb_hbm, src_hbm, dst_hbm, ex_hbm, dinv_hbm, out_hbm,
          sidx_v, didx_v, gidx_v, didx2_v, exq, dq, aq, xcv, cv, out_s, sem):
        c = lax.axis_index("c")
        s = lax.axis_index("s")
        wid = c * NS + s
        zero = _zero16()

        def zfill(i, carry):
            for j in range(C // 16):
                cv[i, pl.ds(16 * j, 16)] = zero
            return carry

        lax.fori_loop(0, G2, zfill, 0)
        r0 = pl.multiple_of(s * (NP // NS), NP // NS)
        for t in range(RP2):
            pltpu.sync_copy(cv, out_s.at[pl.ds(r0 + G2 * t, G2)])
        plsc.subcore_barrier()

        def body(kk, carry):
            base = wid * EPT2 + kk * G2
            pltpu.sync_copy(src_hbm.at[pl.ds(base, G2)], sidx_v)
            pltpu.sync_copy(dst_hbm.at[pl.ds(base, G2)], didx_v)
            for w in range(G2 // 16):
                gidx_v[pl.ds(16 * w, 16)] = sidx_v[pl.ds(16 * w, 16)] + hc * NP
                didx2_v[pl.ds(16 * w, 16)] = didx_v[pl.ds(16 * w, 16)] * heads + hc
            d1 = pltpu.async_copy(xlcb_hbm.at[gidx_v], xcv, sem)
            d2 = pltpu.async_copy(dinv_hbm.at[didx2_v], dq, sem)
            pltpu.sync_copy(ex_hbm.at[hc, pl.ds(base, G2)], exq)
            d1.wait()
            d2.wait()
            for w in range(G2 // 16):
                aq[pl.ds(16 * w, 16)] = exq[pl.ds(16 * w, 16)] * dq[pl.ds(16 * w, 16)]

            def edge(i, carry2):
                av = jnp.full((16,), aq[pl.ds(i, 16)][0], jnp.float32)
                for j in range(C // 16):
                    cv[i, pl.ds(16 * j, 16)] = xcv[i, pl.ds(16 * j, 16)] * av
                return carry2

            lax.fori_loop(0, G2, edge, 0)
            pltpu.sync_copy(cv, out_s.at[didx_v], add=True)
            return carry

        lax.fori_loop(0, K2, body, 0)
        plsc.subcore_barrier()
        for t in range(RP2):
            pltpu.sync_copy(out_s.at[pl.ds(r0 + G2 * t, G2)], cv)
            pltpu.sync_copy(cv, out_hbm.at[c, pl.ds(r0 + G2 * t, G2)])

    return k(xlcb, src2, dst2, ex, deninv)


def _dual_mm_kernel(heads, x_ref, wl_ref, bl_ref, wr_ref, br_ref,
                    o1_ref, o1c_ref, o2_ref):
    xb = x_ref[...]
    chunks = []
    for h in range(heads):
        ch = (jnp.dot(xb, wl_ref[:, pl.ds(h * C, C)],
                      preferred_element_type=jnp.float32)
              + bl_ref[pl.ds(h * C, C)])
        o1c_ref[h] = ch
        chunks.append(ch)
    o1_ref[...] = jnp.concatenate(chunks, axis=1) if heads > 1 else chunks[0]
    o2_ref[...] = jnp.dot(xb, wr_ref[...], preferred_element_type=jnp.float32) + br_ref[...]


def _dual_mm(x, wl, bl, wr, br, bm, heads):
    m, kdim = x.shape
    n = wl.shape[1]
    return pl.pallas_call(
        functools.partial(_dual_mm_kernel, heads),
        grid=(m // bm,),
        in_specs=[pl.BlockSpec((bm, kdim), lambda i: (i, 0)),
                  pl.BlockSpec((kdim, n), lambda i: (0, 0)),
                  pl.BlockSpec((n,), lambda i: (0,)),
                  pl.BlockSpec((kdim, n), lambda i: (0, 0)),
                  pl.BlockSpec((n,), lambda i: (0,))],
        out_specs=[pl.BlockSpec((bm, n), lambda i: (i, 0)),
                   pl.BlockSpec((heads, bm, C), lambda i: (0, i, 0)),
                   pl.BlockSpec((bm, n), lambda i: (i, 0))],
        out_shape=[jax.ShapeDtypeStruct((m, n), jnp.float32),
                   jax.ShapeDtypeStruct((heads, m, C), jnp.float32),
                   jax.ShapeDtypeStruct((m, n), jnp.float32)],
    )(x, wl, bl, wr, br)


def _e_mm_kernel(ea_ref, we_ref, o_ref):
    o_ref[...] = jnp.dot(ea_ref[...], we_ref[...], preferred_element_type=jnp.float32)


def _e_mm(ea2, we):
    m = ea2.shape[0]
    n = we.shape[1]
    bm = 4096
    return pl.pallas_call(
        _e_mm_kernel,
        grid=(m // bm,),
        in_specs=[pl.BlockSpec((bm, ED), lambda i: (i, 0)),
                  pl.BlockSpec((ED, n), lambda i: (0, 0))],
        out_specs=pl.BlockSpec((bm, n), lambda i: (i, 0)),
        out_shape=jax.ShapeDtypeStruct((m, n), jnp.float32),
    )(ea2, we)


def _deninv_kernel(d_ref, o_ref):
    o_ref[...] = 1.0 / (d_ref[0] + d_ref[1] + 1e-16)


def _deninv(dden):
    nph = dden.shape[1]
    d3 = dden.reshape(NC, nph // 128, 128)
    out = pl.pallas_call(
        _deninv_kernel,
        out_shape=jax.ShapeDtypeStruct((nph // 128, 128), jnp.float32),
    )(d3)
    return out.reshape(nph)


def _combine_elu_kernel(p_ref, b_ref, o_ref):
    z = p_ref[0, 0] + p_ref[0, 1] + b_ref[0]
    o_ref[0] = jnp.where(z > 0, z, jnp.exp(z) - 1.0)


def _combine_elu(parts, bias_hc, heads):
    bn = 512
    return pl.pallas_call(
        _combine_elu_kernel,
        grid=(heads, NP // bn),
        in_specs=[pl.BlockSpec((1, NC, bn, C), lambda h, i: (h, 0, i, 0)),
                  pl.BlockSpec((1, 1, C), lambda h, i: (h, 0, 0))],
        out_specs=pl.BlockSpec((1, bn, C), lambda h, i: (h, i, 0)),
        out_shape=jax.ShapeDtypeStruct((heads, NP, C), jnp.float32),
    )(parts, bias_hc.reshape(heads, 1, C))


def _l2_mm_kernel(h_ref, wl_ref, bl_ref, wr_ref, br_ref, o1_ref, o2_ref):
    accl = jnp.zeros(o1_ref.shape, jnp.float32)
    accr = jnp.zeros(o2_ref.shape, jnp.float32)
    for h in range(H):
        hb = h_ref[h]
        accl = accl + jnp.dot(hb, wl_ref[h], preferred_element_type=jnp.float32)
        accr = accr + jnp.dot(hb, wr_ref[h], preferred_element_type=jnp.float32)
    o1_ref[...] = accl + bl_ref[...]
    o2_ref[...] = accr + br_ref[...]


def _l2_mm(h_chunks, wl2, bl2, wr2, br2):
    bn = 512
    wl3 = wl2.reshape(H, C, C)
    wr3 = wr2.reshape(H, C, C)
    return pl.pallas_call(
        _l2_mm_kernel,
        grid=(NP // bn,),
        in_specs=[pl.BlockSpec((H, bn, C), lambda i: (0, i, 0)),
                  pl.BlockSpec((H, C, C), lambda i: (0, 0, 0)),
                  pl.BlockSpec((C,), lambda i: (0,)),
                  pl.BlockSpec((H, C, C), lambda i: (0, 0, 0)),
                  pl.BlockSpec((C,), lambda i: (0,))],
        out_specs=[pl.BlockSpec((bn, C), lambda i: (i, 0)),
                   pl.BlockSpec((bn, C), lambda i: (i, 0))],
        out_shape=[jax.ShapeDtypeStruct((NP, C), jnp.float32),
                   jax.ShapeDtypeStruct((NP, C), jnp.float32)],
    )(h_chunks, wl3, bl2, wr3, br2)


def _mlp_kernel(va_ref, vb_ref, fc1_w_ref, fc1_b_ref, bn_s_ref, bn_o_ref,
                fc2_w_ref, fc2_b_ref, out_w_ref, out_b_ref, o_ref):
    va = va_ref[...]
    vb = vb_ref[...]
    combined = jnp.concatenate([va + vb, jnp.abs(va - vb), va * vb], axis=1)
    z = jnp.dot(combined, fc1_w_ref[...], preferred_element_type=jnp.float32) + fc1_b_ref[...]
    z = z * bn_s_ref[...] + bn_o_ref[...]
    z = jax.nn.relu(z)
    z = jax.nn.relu(jnp.dot(z, fc2_w_ref[...], preferred_element_type=jnp.float32) + fc2_b_ref[...])
    o_ref[...] = jnp.dot(z, out_w_ref[...], preferred_element_type=jnp.float32) + out_b_ref[...]


def _gat_layer(xpad, src2, dst2, e_full, att, bias, wl, bl, wr, br, heads):
    hc_dim = heads * C
    xl, xlc, xr = _dual_mm(xpad, wl, bl, wr, br, 512, heads)
    att_flat = att.reshape(hc_dim)
    ex, dden = _pass1_call(xl, xr, e_full, src2, dst2, att_flat, heads, hc_dim)
    dinv = _deninv(dden)
    xlcb = xlc.reshape(heads * NP, C)
    parts = []
    for hc in range(heads):
        parts.append(_pass2_call(xlcb, src2, dst2, ex, dinv, heads, hc))
    parts = jnp.stack(parts)
    return _combine_elu(parts, bias.reshape(heads, C), heads)


def _arm(x, edge_index, edge_attr, batch, p):
    src, dst = edge_index[0], edge_index[1]
    easum, deg16 = _seg_attr_call(dst, edge_attr)
    deg = deg16[0, :N, 0] + deg16[1, :N, 0]
    loop_attr = (easum[0, :N] + easum[1, :N]) / jnp.maximum(deg, 1.0)[:, None]

    ar = jnp.arange(N, dtype=jnp.int32)
    npad = E2P - E2
    src2 = jnp.concatenate([src, ar, jnp.zeros((npad,), jnp.int32)])
    dst2 = jnp.concatenate([dst, ar, jnp.full((npad,), N, jnp.int32)])
    ea2 = jnp.concatenate([edge_attr, loop_attr, jnp.zeros((npad, ED), jnp.float32)], axis=0)
    xpad = jnp.pad(x, ((0, NP - N), (0, 0)))

    e1 = _e_mm(ea2, p['We1'])
    h1 = _gat_layer(xpad, src2, dst2, e1, p['att1'], p['bias1'],
                    p['Wl1'], p['bl1'], p['Wr1'], p['br1'], H)

    xl2, xr2 = _l2_mm(h1, p['Wl2'], p['bl2'], p['Wr2'], p['br2'])
    e2 = _e_mm(ea2, p['We2'])
    att2_flat = p['att2'].reshape(C)
    ex2, dden2 = _pass1_call(xl2, xr2, e2, src2, dst2, att2_flat, 1, C)
    dinv2 = _deninv(dden2)
    part2 = _pass2_call(xl2, src2, dst2, ex2, dinv2, 1, 0)
    h2 = _combine_elu(part2[None], p['bias2'].reshape(1, C), 1)[0]

    hN = h2[:N]
    cnt = jax.ops.segment_sum(jnp.ones((N,), hN.dtype), batch, num_segments=B)
    mean = jax.ops.segment_sum(hN, batch, num_segments=B) / jnp.maximum(cnt, 1.0)[:, None]
    mx = jax.ops.segment_max(hN, batch, num_segments=B)
    mx = jnp.where(jnp.isfinite(mx), mx, 0.0)
    return jnp.concatenate([mean, mx], axis=1)


def kernel(x_a, edge_index_a, edge_attr_a, batch_a, x_b, edge_index_b, edge_attr_b, batch_b,
           Wl1, bl1, Wr1, br1, We1, att1, bias1, Wl2, bl2, Wr2, br2, We2, att2, bias2,
           fc1_w, fc1_b, bn_g, bn_b, bn_rm, bn_rv, fc2_w, fc2_b, out_w, out_b):
    p = dict(Wl1=Wl1, bl1=bl1, Wr1=Wr1, br1=br1, We1=We1, att1=att1, bias1=bias1,
             Wl2=Wl2, bl2=bl2, Wr2=Wr2, br2=br2, We2=We2, att2=att2, bias2=bias2)
    va = _arm(x_a, edge_index_a, edge_attr_a, batch_a, p)
    vb = _arm(x_b, edge_index_b, edge_attr_b, batch_b, p)
    bn_scale = bn_g / jnp.sqrt(bn_rv + 1e-5)
    bn_off = bn_b - bn_rm * bn_scale
    return pl.pallas_call(
        _mlp_kernel,
        out_shape=jax.ShapeDtypeStruct((B, 1), jnp.float32),
    )(va, vb, fc1_w, fc1_b, bn_scale, bn_off, fc2_w, fc2_b, out_w, out_b)


# speedup vs baseline: 10.9718x; 1.0818x over previous
import functools

import jax
import jax.numpy as jnp
from jax import lax
from jax.experimental import pallas as pl
from jax.experimental.pallas import tpu as pltpu
from jax.experimental.pallas import tpu_sc as plsc

N = 10000
E = 320000
B = 256
D = 128
ED = 16
H = 4
C = 128

NC = 2
NS = 16
NW = NC * NS

NP = 10240
EPT = E // NW
GA = 80
KA = EPT // GA

E2 = E + N
E2P = 331776
EPT2 = E2P // NW
G2 = 128
K2 = EPT2 // G2

_SC_PARAMS = pltpu.CompilerParams(use_tc_tiling_on_sc=False,
                                  needs_layout_passes=False)
_MESH = dict(core_axis_name="c", subcore_axis_name="s")


def _zero16():
    return jnp.zeros((16,), jnp.float32)


def _seg_attr_call(dst, ea):
    ones16 = jnp.zeros((GA, 16), jnp.float32).at[:, 0].set(1.0)
    RPS = NP // NS

    @functools.partial(
        pl.kernel,
        out_type=[jax.ShapeDtypeStruct((NC, NP, 16), jnp.float32),
                  jax.ShapeDtypeStruct((NC, NP, 16), jnp.float32)],
        mesh=plsc.VectorSubcoreMesh(**_MESH),
        compiler_params=_SC_PARAMS,
        scratch_types=[
            pltpu.VMEM((GA,), jnp.int32),
            pltpu.VMEM((GA, 16), jnp.float32),
            pltpu.VMEM((GA, 16), jnp.float32),
            pltpu.VMEM((RPS, 16), jnp.float32),
            pltpu.VMEM_SHARED((NP, 16), jnp.float32),
            pltpu.VMEM_SHARED((NP, 16), jnp.float32),
        ],
    )
    def k(dst_hbm, ea_hbm, ones_hbm, easum_hbm, deg_hbm,
          idx_v, val_v, ones_v, zv, acc_s, dacc_s):
        c = lax.axis_index("c")
        s = lax.axis_index("s")
        wid = c * NS + s
        zero = _zero16()

        def zfill(i, carry):
            zv[i, :] = zero
            return carry

        lax.fori_loop(0, RPS, zfill, 0)
        r0 = pl.multiple_of(s * RPS, RPS)
        pltpu.sync_copy(ones_hbm, ones_v)
        pltpu.sync_copy(zv, acc_s.at[pl.ds(r0, RPS)])
        pltpu.sync_copy(zv, dacc_s.at[pl.ds(r0, RPS)])
        plsc.subcore_barrier()

        def body(kk, carry):
            base = wid * EPT + kk * GA
            pltpu.sync_copy(dst_hbm.at[pl.ds(base, GA)], idx_v)
            pltpu.sync_copy(ea_hbm.at[pl.ds(base, GA)], val_v)
            pltpu.sync_copy(val_v, acc_s.at[idx_v], add=True)
            pltpu.sync_copy(ones_v, dacc_s.at[idx_v], add=True)
            return carry

        lax.fori_loop(0, KA, body, 0)
        plsc.subcore_barrier()
        pltpu.sync_copy(acc_s.at[pl.ds(r0, RPS)], zv)
        pltpu.sync_copy(zv, easum_hbm.at[c, pl.ds(r0, RPS)])
        pltpu.sync_copy(dacc_s.at[pl.ds(r0, RPS)], zv)
        pltpu.sync_copy(zv, deg_hbm.at[c, pl.ds(r0, RPS)])

    return k(dst, ea, ones16)


def _pass1_call(xl, xr, e, src2, dst2, att_flat, heads, hc_dim):
    G1 = 32 if hc_dim >= 512 else 64
    K1 = EPT2 // G1
    NPH = NP * heads
    CHK = NPH // NS
    CPH16 = (hc_dim // heads) // 16

    @functools.partial(
        pl.kernel,
        out_type=[jax.ShapeDtypeStruct((heads, E2P), jnp.float32),
                  jax.ShapeDtypeStruct((NC, NPH), jnp.float32)],
        mesh=plsc.VectorSubcoreMesh(**_MESH),
        compiler_params=_SC_PARAMS,
        scratch_types=[
            pltpu.VMEM((2, G1), jnp.int32),
            pltpu.VMEM((2, G1), jnp.int32),
            pltpu.VMEM((2, G1, hc_dim), jnp.float32),
            pltpu.VMEM((2, G1, hc_dim), jnp.float32),
            pltpu.VMEM((2, G1, hc_dim), jnp.float32),
            pltpu.VMEM((hc_dim,), jnp.float32),
            pltpu.VMEM((heads * G1,), jnp.float32),
            pltpu.VMEM((heads * G1,), jnp.int32),
            pltpu.VMEM((CHK,), jnp.float32),
            pltpu.VMEM_SHARED((NPH,), jnp.float32),
            pltpu.SemaphoreType.DMA,
            pltpu.SemaphoreType.DMA,
        ],
    )
    def k(xl_hbm, xr_hbm, e_hbm, src_hbm, dst_hbm, att_hbm, ex_hbm, den_hbm,
          sidx_v, didx_v, xlv, xrv, ev, attv, exv, fidx_v, zv, den_s,
          sem0, sem1):
        c = lax.axis_index("c")
        s = lax.axis_index("s")
        wid = c * NS + s
        zero = _zero16()
        sems = (sem0, sem1)

        def zfill(i, carry):
            zv[pl.ds(i * 16, 16)] = zero
            return carry

        lax.fori_loop(0, CHK // 16, zfill, 0)
        r0 = pl.multiple_of(s * CHK, CHK)
        pltpu.sync_copy(att_hbm, attv)
        pltpu.sync_copy(zv, den_s.at[pl.ds(r0, CHK)])
        plsc.subcore_barrier()

        def issue(kk, b):
            base = wid * EPT2 + kk * G1
            pltpu.sync_copy(src_hbm.at[pl.ds(base, G1)], sidx_v.at[b])
            pltpu.sync_copy(dst_hbm.at[pl.ds(base, G1)], didx_v.at[b])
            pltpu.async_copy(xl_hbm.at[sidx_v.at[b]], xlv.at[b], sems[b])
            pltpu.async_copy(xr_hbm.at[didx_v.at[b]], xrv.at[b], sems[b])
            pltpu.async_copy(e_hbm.at[pl.ds(base, G1)], ev.at[b], sems[b])

        def wait(b):
            pltpu.make_async_copy(xl_hbm.at[sidx_v.at[b]], xlv.at[b], sems[b]).wait()
            pltpu.make_async_copy(xr_hbm.at[didx_v.at[b]], xrv.at[b], sems[b]).wait()
            pltpu.make_async_copy(e_hbm.at[pl.ds(0, G1)], ev.at[b], sems[b]).wait()

        lane = lax.iota(jnp.int32, 16)

        def compute(kk, b):
            base = wid * EPT2 + kk * G1

            def group(g, carry2):
                def edge16(ii, lvecs):
                    i = g * 16 + ii
                    sel = lane == ii
                    out = []
                    for h in range(heads):
                        acc = _zero16()
                        for jj in range(CPH16):
                            j = h * CPH16 + jj
                            v = (xlv[b, i, pl.ds(16 * j, 16)]
                                 + xrv[b, i, pl.ds(16 * j, 16)]
                                 + ev[b, i, pl.ds(16 * j, 16)])
                            m = jnp.maximum(v, 0.0) + 0.2 * jnp.minimum(v, 0.0)
                            acc = acc + m * attv[pl.ds(16 * j, 16)]
                        sc = jnp.full((16,), jnp.sum(acc), jnp.float32)
                        out.append(jnp.where(sel, sc, lvecs[h]))
                    return tuple(out)

                lvecs = lax.fori_loop(0, 16, edge16,
                                      tuple(_zero16() for _ in range(heads)))
                for h in range(heads):
                    exv[pl.ds(h * G1 + g * 16, 16)] = lvecs[h]
                return carry2

            lax.fori_loop(0, G1 // 16, group, 0)

            for w in range(heads * G1 // 16):
                x = exv[pl.ds(16 * w, 16)]
                exv[pl.ds(16 * w, 16)] = jnp.exp(jnp.minimum(x, 50.0))
                h = (16 * w) // G1
                sub = w - h * (G1 // 16)
                d16 = didx_v[b, pl.ds(16 * sub, 16)]
                fidx_v[pl.ds(16 * w, 16)] = d16 * heads + h
            for h in range(heads):
                pltpu.sync_copy(exv.at[pl.ds(h * G1, G1)],
                                ex_hbm.at[h, pl.ds(base, G1)])
            pltpu.sync_copy(exv, den_s.at[fidx_v], add=True)

        issue(0, 0)
        issue(1, 1)

        def outer(g, carry):
            for b in range(2):
                kk = 2 * g + b
                wait(b)
                compute(kk, b)

                @pl.when(kk + 2 < K1)
                def _():
                    issue(kk + 2, b)
            return carry

        lax.fori_loop(0, K1 // 2, outer, 0)
        plsc.subcore_barrier()
        pltpu.sync_copy(den_s.at[pl.ds(r0, CHK)], zv)
        pltpu.sync_copy(zv, den_hbm.at[c, pl.ds(r0, CHK)])

    return k(xl, xr, e, src2, dst2, att_flat)


def _pass2_call(xlcb, src2, dst2, ex, deninv, heads, hc):
    NPH = NP * heads
    GP = 64
    KP = EPT2 // GP
    RP2 = NP // NS // GP

    @functools.partial(
        pl.kernel,
        out_type=jax.ShapeDtypeStruct((NC, NP, C), jnp.float32),
        mesh=plsc.VectorSubcoreMesh(**_MESH),
        compiler_params=_SC_PARAMS,
        scratch_types=[
            pltpu.VMEM((2, GP), jnp.int32),
            pltpu.VMEM((2, GP), jnp.int32),
            pltpu.VMEM((2, GP), jnp.int32),
            pltpu.VMEM((2, GP), jnp.int32),
            pltpu.VMEM((2, GP), jnp.float32),
            pltpu.VMEM((2, GP), jnp.float32),
            pltpu.VMEM((GP + 16,), jnp.float32),
            pltpu.VMEM((2, GP, C), jnp.float32),
            pltpu.VMEM((GP, C), jnp.float32),
            pltpu.VMEM_SHARED((NP, C), jnp.float32),
            pltpu.SemaphoreType.DMA,
            pltpu.SemaphoreType.DMA,
        ],
    )
    def k(xlcb_hbm, src_hbm, dst_hbm, ex_hbm, dinv_hbm, out_hbm,
          sidx_v, didx_v, gidx_v, didx2_v, exq, dq, aq, xcv, cv, out_s,
          sem0, sem1):
        c = lax.axis_index("c")
        s = lax.axis_index("s")
        wid = c * NS + s
        zero = _zero16()
        sems = (sem0, sem1)

        def zfill(i, carry):
            for j in range(C // 16):
                cv[i, pl.ds(16 * j, 16)] = zero
            return carry

        lax.fori_loop(0, GP, zfill, 0)
        r0 = pl.multiple_of(s * (NP // NS), NP // NS)
        for tch in range(RP2):
            pltpu.sync_copy(cv, out_s.at[pl.ds(r0 + GP * tch, GP)])
        plsc.subcore_barrier()

        def issue(kk, b):
            base = wid * EPT2 + kk * GP
            pltpu.sync_copy(src_hbm.at[pl.ds(base, GP)], sidx_v.at[b])
            pltpu.sync_copy(dst_hbm.at[pl.ds(base, GP)], didx_v.at[b])
            for w in range(GP // 16):
                gidx_v[b, pl.ds(16 * w, 16)] = sidx_v[b, pl.ds(16 * w, 16)] + hc * NP
                didx2_v[b, pl.ds(16 * w, 16)] = didx_v[b, pl.ds(16 * w, 16)] * heads + hc
            pltpu.async_copy(xlcb_hbm.at[gidx_v.at[b]], xcv.at[b], sems[b])
            pltpu.async_copy(dinv_hbm.at[didx2_v.at[b]], dq.at[b], sems[b])
            pltpu.async_copy(ex_hbm.at[hc, pl.ds(base, GP)], exq.at[b], sems[b])

        def wait(b):
            pltpu.make_async_copy(xlcb_hbm.at[gidx_v.at[b]], xcv.at[b], sems[b]).wait()
            pltpu.make_async_copy(dinv_hbm.at[didx2_v.at[b]], dq.at[b], sems[b]).wait()
            pltpu.make_async_copy(ex_hbm.at[hc, pl.ds(0, GP)], exq.at[b], sems[b]).wait()

        def compute(kk, b):
            for w in range(GP // 16):
                aq[pl.ds(16 * w, 16)] = (exq[b, pl.ds(16 * w, 16)]
                                         * dq[b, pl.ds(16 * w, 16)])

            def edge(i, carry2):
                av = jnp.full((16,), aq[pl.ds(i, 16)][0], jnp.float32)
                for j in range(C // 16):
                    cv[i, pl.ds(16 * j, 16)] = xcv[b, i, pl.ds(16 * j, 16)] * av
                return carry2

            lax.fori_loop(0, GP, edge, 0)
            pltpu.sync_copy(cv, out_s.at[didx_v.at[b]], add=True)

        issue(0, 0)
        issue(1, 1)

        def outer(g, carry):
            for b in range(2):
                kk = 2 * g + b
                wait(b)
                compute(kk, b)

                @pl.when(kk + 2 < KP)
                def _():
                    issue(kk + 2, b)
            return carry

        lax.fori_loop(0, KP // 2, outer, 0)
        plsc.subcore_barrier()
        for tch in range(RP2):
            pltpu.sync_copy(out_s.at[pl.ds(r0 + GP * tch, GP)], cv)
            pltpu.sync_copy(cv, out_hbm.at[c, pl.ds(r0 + GP * tch, GP)])

    return k(xlcb, src2, dst2, ex, deninv)


def _dual_mm_kernel(heads, x_ref, wl_ref, bl_ref, wr_ref, br_ref,
                    o1_ref, o1c_ref, o2_ref):
    xb = x_ref[...]
    chunks = []
    for h in range(heads):
        ch = (jnp.dot(xb, wl_ref[:, pl.ds(h * C, C)],
                      preferred_element_type=jnp.float32)
              + bl_ref[pl.ds(h * C, C)])
        o1c_ref[h] = ch
        chunks.append(ch)
    o1_ref[...] = jnp.concatenate(chunks, axis=1) if heads > 1 else chunks[0]
    o2_ref[...] = jnp.dot(xb, wr_ref[...], preferred_element_type=jnp.float32) + br_ref[...]


def _dual_mm(x, wl, bl, wr, br, bm, heads):
    m, kdim = x.shape
    n = wl.shape[1]
    return pl.pallas_call(
        functools.partial(_dual_mm_kernel, heads),
        grid=(m // bm,),
        in_specs=[pl.BlockSpec((bm, kdim), lambda i: (i, 0)),
                  pl.BlockSpec((kdim, n), lambda i: (0, 0)),
                  pl.BlockSpec((n,), lambda i: (0,)),
                  pl.BlockSpec((kdim, n), lambda i: (0, 0)),
                  pl.BlockSpec((n,), lambda i: (0,))],
        out_specs=[pl.BlockSpec((bm, n), lambda i: (i, 0)),
                   pl.BlockSpec((heads, bm, C), lambda i: (0, i, 0)),
                   pl.BlockSpec((bm, n), lambda i: (i, 0))],
        out_shape=[jax.ShapeDtypeStruct((m, n), jnp.float32),
                   jax.ShapeDtypeStruct((heads, m, C), jnp.float32),
                   jax.ShapeDtypeStruct((m, n), jnp.float32)],
    )(x, wl, bl, wr, br)


def _e_mm_kernel(ea_ref, we_ref, o_ref):
    o_ref[...] = jnp.dot(ea_ref[...], we_ref[...], preferred_element_type=jnp.float32)


def _e_mm(ea2, we):
    m = ea2.shape[0]
    n = we.shape[1]
    bm = 4096
    return pl.pallas_call(
        _e_mm_kernel,
        grid=(m // bm,),
        in_specs=[pl.BlockSpec((bm, ED), lambda i: (i, 0)),
                  pl.BlockSpec((ED, n), lambda i: (0, 0))],
        out_specs=pl.BlockSpec((bm, n), lambda i: (i, 0)),
        out_shape=jax.ShapeDtypeStruct((m, n), jnp.float32),
    )(ea2, we)


def _deninv_kernel(d_ref, o_ref):
    o_ref[...] = 1.0 / (d_ref[0] + d_ref[1] + 1e-16)


def _deninv(dden):
    nph = dden.shape[1]
    d3 = dden.reshape(NC, nph // 128, 128)
    out = pl.pallas_call(
        _deninv_kernel,
        out_shape=jax.ShapeDtypeStruct((nph // 128, 128), jnp.float32),
    )(d3)
    return out.reshape(nph)


def _combine_elu_kernel(p_ref, b_ref, o_ref):
    z = p_ref[0, 0] + p_ref[0, 1] + b_ref[0]
    o_ref[0] = jnp.where(z > 0, z, jnp.exp(z) - 1.0)


def _combine_elu(parts, bias_hc, heads):
    bn = 512
    return pl.pallas_call(
        _combine_elu_kernel,
        grid=(heads, NP // bn),
        in_specs=[pl.BlockSpec((1, NC, bn, C), lambda h, i: (h, 0, i, 0)),
                  pl.BlockSpec((1, 1, C), lambda h, i: (h, 0, 0))],
        out_specs=pl.BlockSpec((1, bn, C), lambda h, i: (h, i, 0)),
        out_shape=jax.ShapeDtypeStruct((heads, NP, C), jnp.float32),
    )(parts, bias_hc.reshape(heads, 1, C))


def _l2_mm_kernel(h_ref, wl_ref, bl_ref, wr_ref, br_ref, o1_ref, o2_ref):
    accl = jnp.zeros(o1_ref.shape, jnp.float32)
    accr = jnp.zeros(o2_ref.shape, jnp.float32)
    for h in range(H):
        hb = h_ref[h]
        accl = accl + jnp.dot(hb, wl_ref[h], preferred_element_type=jnp.float32)
        accr = accr + jnp.dot(hb, wr_ref[h], preferred_element_type=jnp.float32)
    o1_ref[...] = accl + bl_ref[...]
    o2_ref[...] = accr + br_ref[...]


def _l2_mm(h_chunks, wl2, bl2, wr2, br2):
    bn = 512
    wl3 = wl2.reshape(H, C, C)
    wr3 = wr2.reshape(H, C, C)
    return pl.pallas_call(
        _l2_mm_kernel,
        grid=(NP // bn,),
        in_specs=[pl.BlockSpec((H, bn, C), lambda i: (0, i, 0)),
                  pl.BlockSpec((H, C, C), lambda i: (0, 0, 0)),
                  pl.BlockSpec((C,), lambda i: (0,)),
                  pl.BlockSpec((H, C, C), lambda i: (0, 0, 0)),
                  pl.BlockSpec((C,), lambda i: (0,))],
        out_specs=[pl.BlockSpec((bn, C), lambda i: (i, 0)),
                   pl.BlockSpec((bn, C), lambda i: (i, 0))],
        out_shape=[jax.ShapeDtypeStruct((NP, C), jnp.float32),
                   jax.ShapeDtypeStruct((NP, C), jnp.float32)],
    )(h_chunks, wl3, bl2, wr3, br2)


def _mlp_kernel(va_ref, vb_ref, fc1_w_ref, fc1_b_ref, bn_s_ref, bn_o_ref,
                fc2_w_ref, fc2_b_ref, out_w_ref, out_b_ref, o_ref):
    va = va_ref[...]
    vb = vb_ref[...]
    combined = jnp.concatenate([va + vb, jnp.abs(va - vb), va * vb], axis=1)
    z = jnp.dot(combined, fc1_w_ref[...], preferred_element_type=jnp.float32) + fc1_b_ref[...]
    z = z * bn_s_ref[...] + bn_o_ref[...]
    z = jax.nn.relu(z)
    z = jax.nn.relu(jnp.dot(z, fc2_w_ref[...], preferred_element_type=jnp.float32) + fc2_b_ref[...])
    o_ref[...] = jnp.dot(z, out_w_ref[...], preferred_element_type=jnp.float32) + out_b_ref[...]


def _gat_layer(xpad, src2, dst2, e_full, att, bias, wl, bl, wr, br, heads):
    hc_dim = heads * C
    xl, xlc, xr = _dual_mm(xpad, wl, bl, wr, br, 512, heads)
    att_flat = att.reshape(hc_dim)
    ex, dden = _pass1_call(xl, xr, e_full, src2, dst2, att_flat, heads, hc_dim)
    dinv = _deninv(dden)
    xlcb = xlc.reshape(heads * NP, C)
    parts = []
    for hc in range(heads):
        parts.append(_pass2_call(xlcb, src2, dst2, ex, dinv, heads, hc))
    parts = jnp.stack(parts)
    return _combine_elu(parts, bias.reshape(heads, C), heads)


def _arm(x, edge_index, edge_attr, batch, p):
    src, dst = edge_index[0], edge_index[1]
    easum, deg16 = _seg_attr_call(dst, edge_attr)
    deg = deg16[0, :N, 0] + deg16[1, :N, 0]
    loop_attr = (easum[0, :N] + easum[1, :N]) / jnp.maximum(deg, 1.0)[:, None]

    ar = jnp.arange(N, dtype=jnp.int32)
    npad = E2P - E2
    src2 = jnp.concatenate([src, ar, jnp.zeros((npad,), jnp.int32)])
    dst2 = jnp.concatenate([dst, ar, jnp.full((npad,), N, jnp.int32)])
    ea2 = jnp.concatenate([edge_attr, loop_attr, jnp.zeros((npad, ED), jnp.float32)], axis=0)
    xpad = jnp.pad(x, ((0, NP - N), (0, 0)))

    e1 = _e_mm(ea2, p['We1'])
    h1 = _gat_layer(xpad, src2, dst2, e1, p['att1'], p['bias1'],
                    p['Wl1'], p['bl1'], p['Wr1'], p['br1'], H)

    xl2, xr2 = _l2_mm(h1, p['Wl2'], p['bl2'], p['Wr2'], p['br2'])
    e2 = _e_mm(ea2, p['We2'])
    att2_flat = p['att2'].reshape(C)
    ex2, dden2 = _pass1_call(xl2, xr2, e2, src2, dst2, att2_flat, 1, C)
    dinv2 = _deninv(dden2)
    part2 = _pass2_call(xl2, src2, dst2, ex2, dinv2, 1, 0)
    h2 = _combine_elu(part2[None], p['bias2'].reshape(1, C), 1)[0]

    hN = h2[:N]
    cnt = jax.ops.segment_sum(jnp.ones((N,), hN.dtype), batch, num_segments=B)
    mean = jax.ops.segment_sum(hN, batch, num_segments=B) / jnp.maximum(cnt, 1.0)[:, None]
    mx = jax.ops.segment_max(hN, batch, num_segments=B)
    mx = jnp.where(jnp.isfinite(mx), mx, 0.0)
    return jnp.concatenate([mean, mx], axis=1)


def kernel(x_a, edge_index_a, edge_attr_a, batch_a, x_b, edge_index_b, edge_attr_b, batch_b,
           Wl1, bl1, Wr1, br1, We1, att1, bias1, Wl2, bl2, Wr2, br2, We2, att2, bias2,
           fc1_w, fc1_b, bn_g, bn_b, bn_rm, bn_rv, fc2_w, fc2_b, out_w, out_b):
    p = dict(Wl1=Wl1, bl1=bl1, Wr1=Wr1, br1=br1, We1=We1, att1=att1, bias1=bias1,
             Wl2=Wl2, bl2=bl2, Wr2=Wr2, br2=br2, We2=We2, att2=att2, bias2=bias2)
    va = _arm(x_a, edge_index_a, edge_attr_a, batch_a, p)
    vb = _arm(x_b, edge_index_b, edge_attr_b, batch_b, p)
    bn_scale = bn_g / jnp.sqrt(bn_rv + 1e-5)
    bn_off = bn_b - bn_rm * bn_scale
    return pl.pallas_call(
        _mlp_kernel,
        out_shape=jax.ShapeDtypeStruct((B, 1), jnp.float32),
    )(va, vb, fc1_w, fc1_b, bn_scale, bn_off, fc2_w, fc2_b, out_w, out_b)
